# baseline (speedup 1.0000x reference)
import math

import jax
import jax.numpy as jnp
from jax import lax
from jax.experimental import pallas as pl
from jax.experimental.pallas import tpu as pltpu

_H = 8
_DH = 32
_F = 256
_N = 64
_KNB = 8
_S = 3
_B = 32
_T = 24
_U = int(_S * math.log2(_N))
_SL = 24
_GROUPS = 32
_ROWS = _GROUPS * _N
_HS = _H * _SL


def _body(x_ref, spa_ref, tem_ref, wq_ref, bq_ref, wk_ref, bk_ref,
          wqi_ref, bqi_ref, wki_ref, bki_ref,
          wv_ref, bv_ref, wm_ref, hmask_ref, slot_ref,
          wo_ref, bo_ref, lnw_ref, lnb_ref,
          wff1_ref, bff1_ref, wff2_ref, bff2_ref, out_ref):
    f32 = jnp.float32
    xb = (x_ref[...] + spa_ref[...]) + tem_ref[...]
    q = jnp.dot(xb, wq_ref[...], preferred_element_type=f32) + bq_ref[...]
    k = jnp.dot(xb, wk_ref[...], preferred_element_type=f32) + bk_ref[...]
    v = jnp.dot(xb, wv_ref[...], preferred_element_type=f32) + bv_ref[...]
    qi = jnp.dot(xb, wqi_ref[...], preferred_element_type=f32) + bqi_ref[...]
    ki = jnp.dot(xb, wki_ref[...], preferred_element_type=f32) + bki_ref[...]

    hmask = hmask_ref[...]
    slot = slot_ref[...]
    iota0i = lax.broadcasted_iota(jnp.int32, (_N, _N), 0)
    iota1i = lax.broadcasted_iota(jnp.int32, (_N, _N), 1)

    group_outs = []
    for g in range(_GROUPS):
        r0 = g * _N
        qg = q[r0:r0 + _N, :]
        kg = k[r0:r0 + _N, :]
        vg = v[r0:r0 + _N, :]
        qig = qi[r0:r0 + _N, :]
        kig = ki[r0:r0 + _N, :]
        qks_parts = []
        for kk in range(_KNB):
            ksh = jnp.concatenate([kig[kk + 1:, :], kig[:kk + 1, :]], axis=0)
            s = qig * ksh
            w = _F // 2
            while w >= _H:
                s = s[:, :w] + s[:, w:]
                w //= 2
            qks_parts.append(s)
        qks = jnp.concatenate(qks_parts, axis=1)
        m_g = jnp.dot(qks, wm_ref[...], preferred_element_type=f32)
        m_gT = jnp.transpose(m_g)

        rank_rows = []
        for h in range(_H):
            mh_row = m_gT[h:h + 1, :]
            mh_col = m_g[:, h:h + 1]
            gt = mh_col > mh_row
            eq = mh_col == mh_row
            cmp = jnp.where(gt | (eq & (iota0i < iota1i)), 1.0, 0.0)
            rank = jnp.sum(cmp, axis=0, keepdims=True)
            rank_rows.append(jnp.broadcast_to(rank, (_SL, _N)))
        rank_cat = jnp.concatenate(rank_rows, axis=0)
        p_block = (slot == rank_cat).astype(f32)

        qsel = jnp.dot(p_block, qg, preferred_element_type=f32) * hmask
        s_all = lax.dot_general(qsel, kg, (((1,), (1,)), ((), ())),
                                preferred_element_type=f32) / (_DH ** 0.5)
        attn = jax.nn.softmax(s_all, axis=-1)
        av = jnp.dot(attn, vg, preferred_element_type=f32) * hmask

        sel = jnp.where(slot < float(_U), attn, -1.0)
        minidx_rows = []
        iota_s = lax.broadcasted_iota(jnp.int32, (_SL, _N), 0).astype(f32)
        for h in range(_H):
            sel_h = sel[h * _SL:(h + 1) * _SL, :]
            colmax = jnp.max(sel_h, axis=0, keepdims=True)
            cand = jnp.where(sel_h == colmax, iota_s, float(_N))
            minidx = jnp.min(cand, axis=0, keepdims=True)
            minidx_rows.append(jnp.broadcast_to(minidx, (_SL, _N)))
        minidx_cat = jnp.concatenate(minidx_rows, axis=0)
        e_t = (slot == minidx_cat).astype(f32)
        e_big = jnp.transpose(e_t)
        group_outs.append(
            jnp.dot(e_big, av, preferred_element_type=f32))
    value = jnp.concatenate(group_outs, axis=0)

    val = jnp.dot(value, wo_ref[...], preferred_element_type=f32) + bo_ref[...]
    mu = jnp.mean(val, axis=1, keepdims=True)
    var = jnp.mean((val - mu) ** 2, axis=1, keepdims=True)
    valn = (val - mu) / jnp.sqrt(var + 1e-5) * lnw_ref[...] + lnb_ref[...]
    hmid = jnp.maximum(
        jnp.dot(valn, wff1_ref[...], preferred_element_type=f32) + bff1_ref[...],
        0.0)
    hout = jnp.dot(hmid, wff2_ref[...], preferred_element_type=f32) + bff2_ref[...]
    hout = hout + valn
    mu2 = jnp.mean(hout, axis=1, keepdims=True)
    var2 = jnp.mean((hout - mu2) ** 2, axis=1, keepdims=True)
    out_ref[...] = (hout - mu2) / jnp.sqrt(var2 + 1e-5)


def kernel(x, spa_eigvalue, spa_eigvec, tem_eigvalue, tem_eigvec,
           Wq, bq, Wk, bk, Wv, bv, Wo, bo, Wproj, bproj, ln_w, ln_b,
           Wff1, bff1, Wff2, bff2):
    f32 = jnp.float32
    spa = jnp.tile(spa_eigvec * spa_eigvalue[None, :], (_GROUPS, 1))
    tem = jnp.tile(tem_eigvec * tem_eigvalue[None, :], (_GROUPS, 1))
    x2 = x.reshape(_B * _T * _N, _F)

    j = jnp.arange(_F)
    perm = (j % _H) * _DH + j // _H
    wq_t, wk_t = Wq.T, Wk.T
    wqi, bqi = wq_t[:, perm], bq[perm]
    wki, bki = wk_t[:, perm], bk[perm]

    col = jnp.arange(_KNB * _H)
    wm = jnp.where((col % _H)[:, None] == jnp.arange(_H)[None, :],
                   Wproj[0, col // _H][:, None], 0.0).astype(f32)

    rows = jnp.arange(_HS)
    hmask = ((rows // _SL)[:, None] == (jnp.arange(_F) // _DH)[None, :]
             ).astype(f32)
    slot = (rows % _SL).astype(f32).reshape(_HS, 1)

    row = lambda b: b.reshape(1, _F)

    def full(shape):
        return pl.BlockSpec(shape, lambda i: tuple(0 for _ in shape))

    grid = (_B * _T) // _GROUPS
    out = pl.pallas_call(
        _body,
        grid=(grid,),
        in_specs=[
            pl.BlockSpec((_ROWS, _F), lambda i: (i, 0)),
            full((_ROWS, _F)), full((_ROWS, _F)),
            full((_F, _F)), full((1, _F)),
            full((_F, _F)), full((1, _F)),
            full((_F, _F)), full((1, _F)),
            full((_F, _F)), full((1, _F)),
            full((_F, _F)), full((1, _F)),
            full((_KNB * _H, _H)),
            full((_HS, _F)),
            full((_HS, 1)),
            full((_F, _F)), full((1, _F)),
            full((1, _F)), full((1, _F)),
            full((_F, _F)), full((1, _F)),
            full((_F, _F)), full((1, _F)),
        ],
        out_specs=pl.BlockSpec((_ROWS, _F), lambda i: (i, 0)),
        out_shape=jax.ShapeDtypeStruct((_B * _T * _N, _F), f32),
        compiler_params=pltpu.CompilerParams(
            dimension_semantics=("arbitrary",)),
    )(x2, spa, tem,
      wq_t, row(bq), wk_t, row(bk),
      wqi, row(bqi), wki, row(bki),
      Wv.T, row(bv),
      wm, hmask, slot,
      Wo.T, row(bo), row(ln_w), row(ln_b),
      Wff1.T, row(bff1), Wff2.T, row(bff2))
    return out.reshape(_B, _T, _N, _F)


# speedup vs baseline: 1.0047x; 1.0047x over previous
import math

import jax
import jax.numpy as jnp
from jax import lax
from jax.experimental import pallas as pl
from jax.experimental.pallas import tpu as pltpu

_H = 8
_DH = 32
_F = 256
_N = 64
_KNB = 8
_S = 3
_B = 32
_T = 24
_U = int(_S * math.log2(_N))
_SL = 24
_GROUPS = 32
_ROWS = _GROUPS * _N
_HS = _H * _SL


def _body(x_ref, spa_ref, tem_ref, wq_ref, bq_ref, wk_ref, bk_ref,
          wqi_ref, bqi_ref, wki_ref, bki_ref,
          wv_ref, bv_ref, wm_ref, hmask_ref, slot_ref,
          wo_ref, bo_ref, lnw_ref, lnb_ref,
          wff1_ref, bff1_ref, wff2_ref, bff2_ref, out_ref):
    f32 = jnp.float32
    xb = (x_ref[...] + spa_ref[...]) + tem_ref[...]
    q = jnp.dot(xb, wq_ref[...], preferred_element_type=f32) + bq_ref[...]
    k = jnp.dot(xb, wk_ref[...], preferred_element_type=f32) + bk_ref[...]
    v = jnp.dot(xb, wv_ref[...], preferred_element_type=f32) + bv_ref[...]
    qi = jnp.dot(xb, wqi_ref[...], preferred_element_type=f32) + bqi_ref[...]
    ki = jnp.dot(xb, wki_ref[...], preferred_element_type=f32) + bki_ref[...]

    hmask = hmask_ref[...]
    slot = slot_ref[...]
    iota0i = lax.broadcasted_iota(jnp.int32, (_N, _N), 0)
    iota1i = lax.broadcasted_iota(jnp.int32, (_N, _N), 1)

    rin = lax.broadcasted_iota(jnp.int32, (_ROWS, 1), 0) % _N
    qks_parts = []
    for kk in range(_KNB):
        sh = kk + 1
        a = jnp.concatenate([ki[sh:, :], ki[:sh, :]], axis=0)
        b = jnp.concatenate([ki[sh - _N:, :], ki[:sh - _N, :]], axis=0)
        ksh = jnp.where(rin < (_N - sh), a, b)
        s = qi * ksh
        w = _F // 2
        while w >= _H:
            s = s[:, :w] + s[:, w:]
            w //= 2
        qks_parts.append(s)
    qks = jnp.concatenate(qks_parts, axis=1)

    group_outs = []
    for g in range(_GROUPS):
        r0 = g * _N
        qg = q[r0:r0 + _N, :]
        kg = k[r0:r0 + _N, :]
        vg = v[r0:r0 + _N, :]
        m_g = jnp.dot(qks[r0:r0 + _N, :], wm_ref[...],
                      preferred_element_type=f32)
        m_gT = jnp.transpose(m_g)

        rank_rows = []
        for h in range(_H):
            mh_row = m_gT[h:h + 1, :]
            mh_col = m_g[:, h:h + 1]
            gt = mh_col > mh_row
            eq = mh_col == mh_row
            cmp = jnp.where(gt | (eq & (iota0i < iota1i)), 1.0, 0.0)
            rank = jnp.sum(cmp, axis=0, keepdims=True)
            rank_rows.append(jnp.broadcast_to(rank, (_SL, _N)))
        rank_cat = jnp.concatenate(rank_rows, axis=0)
        p_block = (slot == rank_cat).astype(f32)

        qsel = jnp.dot(p_block, qg, preferred_element_type=f32) * hmask
        s_all = lax.dot_general(qsel, kg, (((1,), (1,)), ((), ())),
                                preferred_element_type=f32) / (_DH ** 0.5)
        attn = jax.nn.softmax(s_all, axis=-1)
        av = jnp.dot(attn, vg, preferred_element_type=f32) * hmask

        sel = jnp.where(slot < float(_U), attn, -1.0)
        minidx_rows = []
        iota_s = lax.broadcasted_iota(jnp.int32, (_SL, _N), 0).astype(f32)
        for h in range(_H):
            sel_h = sel[h * _SL:(h + 1) * _SL, :]
            colmax = jnp.max(sel_h, axis=0, keepdims=True)
            cand = jnp.where(sel_h == colmax, iota_s, float(_N))
            minidx = jnp.min(cand, axis=0, keepdims=True)
            minidx_rows.append(jnp.broadcast_to(minidx, (_SL, _N)))
        minidx_cat = jnp.concatenate(minidx_rows, axis=0)
        e_t = (slot == minidx_cat).astype(f32)
        e_big = jnp.transpose(e_t)
        group_outs.append(
            jnp.dot(e_big, av, preferred_element_type=f32))
    value = jnp.concatenate(group_outs, axis=0)

    val = jnp.dot(value, wo_ref[...], preferred_element_type=f32) + bo_ref[...]
    mu = jnp.mean(val, axis=1, keepdims=True)
    var = jnp.mean((val - mu) ** 2, axis=1, keepdims=True)
    valn = (val - mu) / jnp.sqrt(var + 1e-5) * lnw_ref[...] + lnb_ref[...]
    hmid = jnp.maximum(
        jnp.dot(valn, wff1_ref[...], preferred_element_type=f32) + bff1_ref[...],
        0.0)
    hout = jnp.dot(hmid, wff2_ref[...], preferred_element_type=f32) + bff2_ref[...]
    hout = hout + valn
    mu2 = jnp.mean(hout, axis=1, keepdims=True)
    var2 = jnp.mean((hout - mu2) ** 2, axis=1, keepdims=True)
    out_ref[...] = (hout - mu2) / jnp.sqrt(var2 + 1e-5)


def kernel(x, spa_eigvalue, spa_eigvec, tem_eigvalue, tem_eigvec,
           Wq, bq, Wk, bk, Wv, bv, Wo, bo, Wproj, bproj, ln_w, ln_b,
           Wff1, bff1, Wff2, bff2):
    f32 = jnp.float32
    spa = jnp.tile(spa_eigvec * spa_eigvalue[None, :], (_GROUPS, 1))
    tem = jnp.tile(tem_eigvec * tem_eigvalue[None, :], (_GROUPS, 1))
    x2 = x.reshape(_B * _T * _N, _F)

    j = jnp.arange(_F)
    perm = (j % _H) * _DH + j // _H
    wq_t, wk_t = Wq.T, Wk.T
    wqi, bqi = wq_t[:, perm], bq[perm]
    wki, bki = wk_t[:, perm], bk[perm]

    col = jnp.arange(_KNB * _H)
    wm = jnp.where((col % _H)[:, None] == jnp.arange(_H)[None, :],
                   Wproj[0, col // _H][:, None], 0.0).astype(f32)

    rows = jnp.arange(_HS)
    hmask = ((rows // _SL)[:, None] == (jnp.arange(_F) // _DH)[None, :]
             ).astype(f32)
    slot = (rows % _SL).astype(f32).reshape(_HS, 1)

    row = lambda b: b.reshape(1, _F)

    def full(shape):
        return pl.BlockSpec(shape, lambda i: tuple(0 for _ in shape))

    grid = (_B * _T) // _GROUPS
    out = pl.pallas_call(
        _body,
        grid=(grid,),
        in_specs=[
            pl.BlockSpec((_ROWS, _F), lambda i: (i, 0)),
            full((_ROWS, _F)), full((_ROWS, _F)),
            full((_F, _F)), full((1, _F)),
            full((_F, _F)), full((1, _F)),
            full((_F, _F)), full((1, _F)),
            full((_F, _F)), full((1, _F)),
            full((_F, _F)), full((1, _F)),
            full((_KNB * _H, _H)),
            full((_HS, _F)),
            full((_HS, 1)),
            full((_F, _F)), full((1, _F)),
            full((1, _F)), full((1, _F)),
            full((_F, _F)), full((1, _F)),
            full((_F, _F)), full((1, _F)),
        ],
        out_specs=pl.BlockSpec((_ROWS, _F), lambda i: (i, 0)),
        out_shape=jax.ShapeDtypeStruct((_B * _T * _N, _F), f32),
        compiler_params=pltpu.CompilerParams(
            dimension_semantics=("arbitrary",)),
    )(x2, spa, tem,
      wq_t, row(bq), wk_t, row(bk),
      wqi, row(bqi), wki, row(bki),
      Wv.T, row(bv),
      wm, hmask, slot,
      Wo.T, row(bo), row(ln_w), row(ln_b),
      Wff1.T, row(bff1), Wff2.T, row(bff2))
    return out.reshape(_B, _T, _N, _F)


# speedup vs baseline: 1.0213x; 1.0165x over previous
import math

import jax
import jax.numpy as jnp
from jax import lax
from jax.experimental import pallas as pl
from jax.experimental.pallas import tpu as pltpu

_H = 8
_DH = 32
_F = 256
_N = 64
_KNB = 8
_S = 3
_B = 32
_T = 24
_U = int(_S * math.log2(_N))
_SL = 24
_GROUPS = 32
_ROWS = _GROUPS * _N
_HS = _H * _SL


def _body(x_ref, spa_ref, tem_ref, wqkv_ref, bqkv_ref,
          wm_ref, hmask_ref, slot_ref,
          wo_ref, bo_ref, lnw_ref, lnb_ref,
          wff1_ref, bff1_ref, wff2_ref, bff2_ref, out_ref):
    f32 = jnp.float32
    xb = (x_ref[...] + spa_ref[...]) + tem_ref[...]
    big = jnp.dot(xb, wqkv_ref[...], preferred_element_type=f32) + bqkv_ref[...]
    q = big[:, 0 * _F:1 * _F]
    k = big[:, 1 * _F:2 * _F]
    v = big[:, 2 * _F:3 * _F]
    qi = big[:, 3 * _F:4 * _F]
    ki = big[:, 4 * _F:5 * _F]

    hmask = hmask_ref[...]
    slot = slot_ref[...]
    iota0i = lax.broadcasted_iota(jnp.int32, (_N, _N), 0)
    iota1i = lax.broadcasted_iota(jnp.int32, (_N, _N), 1)

    rin = lax.broadcasted_iota(jnp.int32, (_ROWS, 1), 0) % _N
    qks_parts = []
    for kk in range(_KNB):
        sh = kk + 1
        a = jnp.concatenate([ki[sh:, :], ki[:sh, :]], axis=0)
        b = jnp.concatenate([ki[sh - _N:, :], ki[:sh - _N, :]], axis=0)
        ksh = jnp.where(rin < (_N - sh), a, b)
        s = qi * ksh
        w = _F // 2
        while w >= _H:
            s = s[:, :w] + s[:, w:]
            w //= 2
        qks_parts.append(s)
    qks = jnp.concatenate(qks_parts, axis=1)

    group_outs = []
    for g in range(_GROUPS):
        r0 = g * _N
        qg = q[r0:r0 + _N, :]
        kg = k[r0:r0 + _N, :]
        vg = v[r0:r0 + _N, :]
        m_g = jnp.dot(qks[r0:r0 + _N, :], wm_ref[...],
                      preferred_element_type=f32)
        m_gT = jnp.transpose(m_g)

        rank_rows = []
        for h in range(_H):
            mh_row = m_gT[h:h + 1, :]
            mh_col = m_g[:, h:h + 1]
            gt = mh_col > mh_row
            eq = mh_col == mh_row
            cmp = jnp.where(gt | (eq & (iota0i < iota1i)), 1.0, 0.0)
            rank = jnp.sum(cmp, axis=0, keepdims=True)
            rank_rows.append(jnp.broadcast_to(rank, (_SL, _N)))
        rank_cat = jnp.concatenate(rank_rows, axis=0)
        p_block = (slot == rank_cat).astype(f32)

        qsel = jnp.dot(p_block, qg, preferred_element_type=f32) * hmask
        s_all = lax.dot_general(qsel, kg, (((1,), (1,)), ((), ())),
                                preferred_element_type=f32) / (_DH ** 0.5)
        attn = jax.nn.softmax(s_all, axis=-1)
        av = jnp.dot(attn, vg, preferred_element_type=f32) * hmask

        sel = jnp.where(slot < float(_U), attn, -1.0)
        minidx_rows = []
        iota_s = lax.broadcasted_iota(jnp.int32, (_SL, _N), 0).astype(f32)
        for h in range(_H):
            sel_h = sel[h * _SL:(h + 1) * _SL, :]
            colmax = jnp.max(sel_h, axis=0, keepdims=True)
            cand = jnp.where(sel_h == colmax, iota_s, float(_N))
            minidx = jnp.min(cand, axis=0, keepdims=True)
            minidx_rows.append(jnp.broadcast_to(minidx, (_SL, _N)))
        minidx_cat = jnp.concatenate(minidx_rows, axis=0)
        e_t = (slot == minidx_cat).astype(f32)
        e_big = jnp.transpose(e_t)
        group_outs.append(
            jnp.dot(e_big, av, preferred_element_type=f32))
    value = jnp.concatenate(group_outs, axis=0)

    val = jnp.dot(value, wo_ref[...], preferred_element_type=f32) + bo_ref[...]
    mu = jnp.mean(val, axis=1, keepdims=True)
    var = jnp.mean((val - mu) ** 2, axis=1, keepdims=True)
    valn = (val - mu) / jnp.sqrt(var + 1e-5) * lnw_ref[...] + lnb_ref[...]
    hmid = jnp.maximum(
        jnp.dot(valn, wff1_ref[...], preferred_element_type=f32) + bff1_ref[...],
        0.0)
    hout = jnp.dot(hmid, wff2_ref[...], preferred_element_type=f32) + bff2_ref[...]
    hout = hout + valn
    mu2 = jnp.mean(hout, axis=1, keepdims=True)
    var2 = jnp.mean((hout - mu2) ** 2, axis=1, keepdims=True)
    out_ref[...] = (hout - mu2) / jnp.sqrt(var2 + 1e-5)


def kernel(x, spa_eigvalue, spa_eigvec, tem_eigvalue, tem_eigvec,
           Wq, bq, Wk, bk, Wv, bv, Wo, bo, Wproj, bproj, ln_w, ln_b,
           Wff1, bff1, Wff2, bff2):
    f32 = jnp.float32
    spa = jnp.tile(spa_eigvec * spa_eigvalue[None, :], (_GROUPS, 1))
    tem = jnp.tile(tem_eigvec * tem_eigvalue[None, :], (_GROUPS, 1))
    x2 = x.reshape(_B * _T * _N, _F)

    j = jnp.arange(_F)
    perm = (j % _H) * _DH + j // _H
    wq_t, wk_t = Wq.T, Wk.T
    wqkv = jnp.concatenate(
        [wq_t, wk_t, Wv.T, wq_t[:, perm], wk_t[:, perm]], axis=1)
    bqkv = jnp.concatenate([bq, bk, bv, bq[perm], bk[perm]])

    col = jnp.arange(_KNB * _H)
    wm = jnp.where((col % _H)[:, None] == jnp.arange(_H)[None, :],
                   Wproj[0, col // _H][:, None], 0.0).astype(f32)

    rows = jnp.arange(_HS)
    hmask = ((rows // _SL)[:, None] == (jnp.arange(_F) // _DH)[None, :]
             ).astype(f32)
    slot = (rows % _SL).astype(f32).reshape(_HS, 1)

    row = lambda b: b.reshape(1, _F)

    def full(shape):
        return pl.BlockSpec(shape, lambda i: tuple(0 for _ in shape))

    grid = (_B * _T) // _GROUPS
    out = pl.pallas_call(
        _body,
        grid=(grid,),
        in_specs=[
            pl.BlockSpec((_ROWS, _F), lambda i: (i, 0)),
            full((_ROWS, _F)), full((_ROWS, _F)),
            full((_F, 5 * _F)), full((1, 5 * _F)),
            full((_KNB * _H, _H)),
            full((_HS, _F)),
            full((_HS, 1)),
            full((_F, _F)), full((1, _F)),
            full((1, _F)), full((1, _F)),
            full((_F, _F)), full((1, _F)),
            full((_F, _F)), full((1, _F)),
        ],
        out_specs=pl.BlockSpec((_ROWS, _F), lambda i: (i, 0)),
        out_shape=jax.ShapeDtypeStruct((_B * _T * _N, _F), f32),
        compiler_params=pltpu.CompilerParams(
            dimension_semantics=("arbitrary",)),
    )(x2, spa, tem,
      wqkv, bqkv.reshape(1, 5 * _F),
      wm, hmask, slot,
      Wo.T, row(bo), row(ln_w), row(ln_b),
      Wff1.T, row(bff1), Wff2.T, row(bff2))
    return out.reshape(_B, _T, _N, _F)
